# baseline (speedup 1.0000x reference)
import functools

import jax
import jax.numpy as jnp
from jax import lax
from jax.experimental import pallas as pl
from jax.experimental.pallas import tpu as pltpu
from jax.experimental.pallas import tpu_sc as plsc

_INFO = plsc.get_sparse_core_info()
_NC, _NS = _INFO.num_cores, _INFO.num_subcores
_NW = _NC * _NS
_CHUNK = 50_000


def _stream_range(src, dst, base, total, bufs, isems, osems):
    n = total // _CHUNK

    def icp(k):
        j = k & 1
        return pltpu.make_async_copy(
            src.at[pl.ds(base + k * _CHUNK, _CHUNK)], bufs[j], isems.at[j])

    def ocp(k):
        j = k & 1
        return pltpu.make_async_copy(
            bufs[j], dst.at[pl.ds(base + k * _CHUNK, _CHUNK)], osems.at[j])

    icp(0).start()
    for k in range(n):
        icp(k).wait()
        ocp(k).start()
        if k + 1 < n:
            if k >= 1:
                ocp(k - 1).wait()
            icp(k + 1).start()
    for k in range(max(0, n - 2), n):
        ocp(k).wait()


def _sc_copy(u_flat, i_flat):
    u_n, i_n = u_flat.shape[0], i_flat.shape[0]
    u_per, i_per = u_n // _NW, i_n // _NW
    mesh = plsc.VectorSubcoreMesh(core_axis_name="c", subcore_axis_name="s")

    @functools.partial(
        pl.kernel,
        mesh=mesh,
        out_type=[
            jax.ShapeDtypeStruct((u_n,), jnp.float32),
            jax.ShapeDtypeStruct((i_n,), jnp.float32),
        ],
        scratch_types=[
            pltpu.VMEM((_CHUNK,), jnp.float32),
            pltpu.VMEM((_CHUNK,), jnp.float32),
            pltpu.SemaphoreType.DMA((2,)),
            pltpu.SemaphoreType.DMA((2,)),
        ],
    )
    def k(u_in, i_in, u_out, i_out, buf0, buf1, isems, osems):
        wid = lax.axis_index("s") * _NC + lax.axis_index("c")
        _stream_range(u_in, u_out, wid * u_per, u_per,
                      (buf0, buf1), isems, osems)
        _stream_range(i_in, i_out, wid * i_per, i_per,
                      (buf0, buf1), isems, osems)

    return k(u_flat, i_flat)


def kernel(embed_user, embed_item):
    u_shape, i_shape = embed_user.shape, embed_item.shape
    u_flat = embed_user.T.reshape(-1)
    i_flat = embed_item.T.reshape(-1)
    out_u, out_i = _sc_copy(u_flat, i_flat)
    out_u = out_u.reshape(u_shape[1], u_shape[0]).T
    out_i = out_i.reshape(i_shape[1], i_shape[0]).T
    return (out_u, out_i)


# speedup vs baseline: 69.3378x; 69.3378x over previous
import jax
import jax.numpy as jnp
from jax.experimental import pallas as pl
from jax.experimental.pallas import tpu as pltpu

_BLOCK_COLS = 40960
_BLOCK_COLS_I = 8192


def _make_body(nblk_i):
    def _copy_body(u_ref, i_ref, uo_ref, io_ref):
        uo_ref[...] = u_ref[...]

        @pl.when(pl.program_id(0) < nblk_i)
        def _():
            io_ref[...] = i_ref[...]

    return _copy_body


def kernel(embed_user, embed_item):
    ut = embed_user.T
    it = embed_item.T
    dim, ucols = ut.shape
    icols = it.shape[1]
    nblk_u = (ucols + _BLOCK_COLS - 1) // _BLOCK_COLS
    nblk_i = (icols + _BLOCK_COLS_I - 1) // _BLOCK_COLS_I

    u_spec = pl.BlockSpec((dim, _BLOCK_COLS), lambda j: (0, j))
    i_spec = pl.BlockSpec((dim, _BLOCK_COLS_I),
                          lambda j: (0, jnp.minimum(j, nblk_i - 1)))
    out_u, out_i = pl.pallas_call(
        _make_body(nblk_i),
        grid=(max(nblk_u, nblk_i),),
        in_specs=[u_spec, i_spec],
        out_specs=[u_spec, i_spec],
        out_shape=[
            jax.ShapeDtypeStruct(ut.shape, ut.dtype),
            jax.ShapeDtypeStruct(it.shape, it.dtype),
        ],
        compiler_params=pltpu.CompilerParams(
            dimension_semantics=("arbitrary",),
        ),
    )(ut, it)
    return (out_u.T, out_i.T)


# speedup vs baseline: 69.3545x; 1.0002x over previous
import jax
import jax.numpy as jnp
from jax.experimental import pallas as pl
from jax.experimental.pallas import tpu as pltpu

_BLOCK_COLS = 49152
_BLOCK_COLS_I = 8192


def _make_body(nblk_i):
    def _copy_body(u_ref, i_ref, uo_ref, io_ref):
        uo_ref[...] = u_ref[...]

        @pl.when(pl.program_id(0) < nblk_i)
        def _():
            io_ref[...] = i_ref[...]

    return _copy_body


def kernel(embed_user, embed_item):
    ut = embed_user.T
    it = embed_item.T
    dim, ucols = ut.shape
    icols = it.shape[1]
    nblk_u = (ucols + _BLOCK_COLS - 1) // _BLOCK_COLS
    nblk_i = (icols + _BLOCK_COLS_I - 1) // _BLOCK_COLS_I

    u_spec = pl.BlockSpec((dim, _BLOCK_COLS), lambda j: (0, j))
    i_spec = pl.BlockSpec((dim, _BLOCK_COLS_I),
                          lambda j: (0, jnp.minimum(j, nblk_i - 1)))
    out_u, out_i = pl.pallas_call(
        _make_body(nblk_i),
        grid=(max(nblk_u, nblk_i),),
        in_specs=[u_spec, i_spec],
        out_specs=[u_spec, i_spec],
        out_shape=[
            jax.ShapeDtypeStruct(ut.shape, ut.dtype),
            jax.ShapeDtypeStruct(it.shape, it.dtype),
        ],
        compiler_params=pltpu.CompilerParams(
            dimension_semantics=("arbitrary",),
        ),
    )(ut, it)
    return (out_u.T, out_i.T)
